# baseline (speedup 1.0000x reference)
import functools

import jax
import jax.numpy as jnp
from jax import lax
from jax.experimental import pallas as pl
from jax.experimental.pallas import tpu as pltpu
from jax.experimental.pallas import tpu_sc as plsc

N = 10000
E = 320000
D = 128

NC = 2
NS = 16
NW = NC * NS
C = 128
NCH = 80
E_PAD = NW * NCH * C
N_PAD = 10240
SLAB = N_PAD // NS

_mesh = plsc.VectorSubcoreMesh(core_axis_name="c", subcore_axis_name="s")


def _zero_vmem(ref, nrows, ncol):
    zeros = jnp.zeros((16,), jnp.float32)

    def body(i, _):
        for k in range(ncol // 16):
            ref[i, pl.ds(k * 16, 16)] = zeros
        return 0

    lax.fori_loop(0, nrows, body, 0)


@functools.partial(
    pl.kernel,
    out_type=jax.ShapeDtypeStruct((NC, N_PAD, D), jnp.float32),
    mesh=_mesh,
    scratch_types=[
        pltpu.VMEM((2, C), jnp.int32),
        pltpu.VMEM((C, D), jnp.float32),
        pltpu.VMEM((32, D), jnp.float32),
        pltpu.VMEM_SHARED((N_PAD, D), jnp.float32),
    ],
)
def _sc_degree(idx_hbm, out_hbm, idx_v, ones_v, zb_v, acc_sh):
    c = lax.axis_index("c")
    s = lax.axis_index("s")
    wid = c * NS + s

    one = jnp.full((16,), 1.0, jnp.float32)

    def fill_ones(i, _):
        for k in range(D // 16):
            ones_v[i, pl.ds(k * 16, 16)] = one
        return 0

    lax.fori_loop(0, C, fill_ones, 0)
    _zero_vmem(zb_v, 32, D)
    row0 = s * SLAB
    for k in range(SLAB // 32):
        pltpu.sync_copy(zb_v, acc_sh.at[pl.ds(row0 + k * 32, 32)])
    plsc.subcore_barrier()

    def chunk(j, _):
        pltpu.sync_copy(idx_hbm.at[wid, j], idx_v)
        pltpu.sync_copy(ones_v, acc_sh.at[idx_v.at[1]], add=True)
        return 0

    lax.fori_loop(0, NCH, chunk, 0)
    plsc.subcore_barrier()
    pltpu.sync_copy(acc_sh.at[pl.ds(row0, SLAB)],
                    out_hbm.at[c, pl.ds(row0, SLAB)])


@functools.partial(
    pl.kernel,
    out_type=jax.ShapeDtypeStruct((NC, N_PAD, D), jnp.float32),
    mesh=_mesh,
    scratch_types=[
        pltpu.VMEM((2, C), jnp.int32),
        pltpu.VMEM((2, C), jnp.int32),
        pltpu.VMEM((C, D), jnp.float32),
        pltpu.VMEM((C, D), jnp.float32),
        pltpu.VMEM((32, D), jnp.float32),
        pltpu.VMEM_SHARED((N_PAD, D), jnp.float32),
        pltpu.SemaphoreType.DMA,
        pltpu.SemaphoreType.DMA,
    ],
)
def _sc_aggregate(g_hbm, idx_hbm, out_hbm,
                  idx_a, idx_b, rows_a, rows_b,
                  zb_v, acc_sh, sem_a, sem_b):
    c = lax.axis_index("c")
    s = lax.axis_index("s")
    wid = c * NS + s

    _zero_vmem(zb_v, 32, D)
    row0 = s * SLAB
    for k in range(SLAB // 32):
        pltpu.sync_copy(zb_v, acc_sh.at[pl.ds(row0 + k * 32, 32)])
    plsc.subcore_barrier()

    idx = (idx_a, idx_b)
    rows = (rows_a, rows_b)
    sem = (sem_a, sem_b)

    def start(slot, chunk_j):
        pltpu.sync_copy(idx_hbm.at[wid, chunk_j], idx[slot])
        pltpu.async_copy(g_hbm.at[idx[slot].at[0]], rows[slot], sem[slot])

    def finish(slot):
        pltpu.make_async_copy(g_hbm.at[idx[slot].at[0]], rows[slot],
                              sem[slot]).wait()
        pltpu.sync_copy(rows[slot], acc_sh.at[idx[slot].at[1]], add=True)

    start(0, 0)

    def pair(j, _):
        start(1, 2 * j + 1)
        finish(0)
        start(0, 2 * j + 2)
        finish(1)
        return 0

    lax.fori_loop(0, NCH // 2 - 1, pair, 0)
    start(1, NCH - 1)
    finish(0)
    finish(1)

    plsc.subcore_barrier()
    pltpu.sync_copy(acc_sh.at[pl.ds(row0, SLAB)],
                    out_hbm.at[c, pl.ds(row0, SLAB)])


_R = 2000


def _gelu(z):
    return 0.5 * z * (1.0 + lax.erf(z * 0.7071067811865476))


def _tc_m_body(x_ref, w1_ref, h_ref):
    h_ref[...] = jnp.dot(x_ref[...], w1_ref[...],
                         preferred_element_type=jnp.float32)


def _tc_m(x, W1):
    return pl.pallas_call(
        _tc_m_body,
        grid=(N // _R,),
        in_specs=[
            pl.BlockSpec((_R, D), lambda i: (i, 0)),
            pl.BlockSpec((D, D), lambda i: (0, 0)),
        ],
        out_specs=pl.BlockSpec((_R, D), lambda i: (i, 0)),
        out_shape=jax.ShapeDtypeStruct((N, D), jnp.float32),
    )(x, W1)


def _tc_a_body(h_ref, degp_ref, g1_ref, dinvb_ref):
    d = degp_ref[0, :, 0:1] + degp_ref[1, :, 0:1] + 1.0
    dinv = lax.rsqrt(d)
    g1_ref[...] = h_ref[...] * dinv
    dinvb_ref[...] = jnp.broadcast_to(dinv, (_R, D))


def _tc_a(h1, degp):
    return pl.pallas_call(
        _tc_a_body,
        grid=(N // _R,),
        in_specs=[
            pl.BlockSpec((_R, D), lambda i: (i, 0)),
            pl.BlockSpec((NC, _R, D), lambda i: (0, i, 0)),
        ],
        out_specs=[
            pl.BlockSpec((_R, D), lambda i: (i, 0)),
            pl.BlockSpec((_R, D), lambda i: (i, 0)),
        ],
        out_shape=[
            jax.ShapeDtypeStruct((N, D), jnp.float32),
            jax.ShapeDtypeStruct((N, D), jnp.float32),
        ],
    )(h1, degp)


def _tc_b_body(p_ref, g1_ref, dinvb_ref, b1_ref, w2_ref, g2_ref):
    agg = p_ref[0] + p_ref[1] + g1_ref[...]
    z = agg * dinvb_ref[...] + b1_ref[...][None, :]
    a = _gelu(z)
    h2 = jnp.dot(a, w2_ref[...], preferred_element_type=jnp.float32)
    g2_ref[...] = h2 * dinvb_ref[...]


def _tc_b(p1, g1, dinvb, b1, W2):
    return pl.pallas_call(
        _tc_b_body,
        grid=(N // _R,),
        in_specs=[
            pl.BlockSpec((NC, _R, D), lambda i: (0, i, 0)),
            pl.BlockSpec((_R, D), lambda i: (i, 0)),
            pl.BlockSpec((_R, D), lambda i: (i, 0)),
            pl.BlockSpec((D,), lambda i: (0,)),
            pl.BlockSpec((D, D), lambda i: (0, 0)),
        ],
        out_specs=pl.BlockSpec((_R, D), lambda i: (i, 0)),
        out_shape=jax.ShapeDtypeStruct((N, D), jnp.float32),
    )(p1, g1, dinvb, b1, W2)


def _tc_c_body(p_ref, g2_ref, dinvb_ref, b2_ref, out_ref):
    agg = p_ref[0] + p_ref[1] + g2_ref[...]
    out_ref[...] = agg * dinvb_ref[...] + b2_ref[...][None, :]


def _tc_c(p2, g2, dinvb, b2):
    return pl.pallas_call(
        _tc_c_body,
        grid=(N // _R,),
        in_specs=[
            pl.BlockSpec((NC, _R, D), lambda i: (0, i, 0)),
            pl.BlockSpec((_R, D), lambda i: (i, 0)),
            pl.BlockSpec((_R, D), lambda i: (i, 0)),
            pl.BlockSpec((D,), lambda i: (0,)),
        ],
        out_specs=pl.BlockSpec((_R, D), lambda i: (i, 0)),
        out_shape=jax.ShapeDtypeStruct((N, D), jnp.float32),
    )(p2, g2, dinvb, b2)


def kernel(x, edge_index, W1, b1, W2, b2):
    pad = E_PAD - E
    pad_src = jnp.arange(pad, dtype=jnp.int32) % N
    pad_dst = N + jnp.arange(pad, dtype=jnp.int32) % (N_PAD - N)
    src_p = jnp.concatenate([edge_index[0], pad_src]).reshape(NW, NCH, C)
    dst_p = jnp.concatenate([edge_index[1], pad_dst]).reshape(NW, NCH, C)
    idx3 = jnp.stack([src_p, dst_p], axis=2)

    degp = _sc_degree(idx3)
    h1 = _tc_m(x, W1)
    g1, dinvb = _tc_a(h1, degp)
    p1 = _sc_aggregate(g1, idx3)
    g2 = _tc_b(p1, g1, dinvb, b1, W2)
    p2 = _sc_aggregate(g2, idx3)
    return _tc_c(p2, g2, dinvb, b2)


# speedup vs baseline: 1.0946x; 1.0946x over previous
import functools

import jax
import jax.numpy as jnp
from jax import lax
from jax.experimental import pallas as pl
from jax.experimental.pallas import tpu as pltpu
from jax.experimental.pallas import tpu_sc as plsc

N = 10000
E = 320000
D = 128

NC = 2
NS = 16
NW = NC * NS
C = 128
NCH = 80
E_PAD = NW * NCH * C
N_PAD = 10240
SLAB = N_PAD // NS

_mesh = plsc.VectorSubcoreMesh(core_axis_name="c", subcore_axis_name="s")


def _zero_vmem(ref, nrows, ncol):
    zeros = jnp.zeros((16,), jnp.float32)

    def body(i, _):
        for k in range(ncol // 16):
            ref[i, pl.ds(k * 16, 16)] = zeros
        return 0

    lax.fori_loop(0, nrows, body, 0)


@functools.partial(
    pl.kernel,
    out_type=jax.ShapeDtypeStruct((NC, N_PAD, D), jnp.float32),
    mesh=_mesh,
    scratch_types=[
        pltpu.VMEM((2, C), jnp.int32),
        pltpu.VMEM((2, C), jnp.int32),
        pltpu.VMEM((C, D), jnp.float32),
        pltpu.VMEM((32, D), jnp.float32),
        pltpu.VMEM_SHARED((N_PAD, D), jnp.float32),
        pltpu.SemaphoreType.DMA,
        pltpu.SemaphoreType.DMA,
    ],
)
def _sc_degree(idx_hbm, out_hbm, idx_a, idx_b, ones_v, zb_v, acc_sh,
               semi_a, semi_b):
    c = lax.axis_index("c")
    s = lax.axis_index("s")
    wid = c * NS + s
    idx = (idx_a, idx_b)
    semi = (semi_a, semi_b)

    one = jnp.full((16,), 1.0, jnp.float32)

    def fill_ones(i, _):
        for k in range(D // 16):
            ones_v[i, pl.ds(k * 16, 16)] = one
        return 0

    lax.fori_loop(0, C, fill_ones, 0)
    _zero_vmem(zb_v, 32, D)
    row0 = s * SLAB
    for k in range(SLAB // 32):
        pltpu.sync_copy(zb_v, acc_sh.at[pl.ds(row0 + k * 32, 32)])
    plsc.subcore_barrier()

    def issue_idx(slot, j):
        pltpu.async_copy(idx_hbm.at[wid, j], idx[slot], semi[slot])

    def wait_idx(slot, j):
        pltpu.make_async_copy(idx_hbm.at[wid, j], idx[slot],
                              semi[slot]).wait()

    issue_idx(0, 0)
    issue_idx(1, 1)

    def pair(j, _):
        for b in range(2):
            k = 2 * j + b
            wait_idx(b, k)
            pltpu.sync_copy(ones_v, acc_sh.at[idx[b].at[1]], add=True)
            issue_idx(b, k + 2)
        return 0

    lax.fori_loop(0, NCH // 2 - 1, pair, 0)
    wait_idx(0, NCH - 2)
    pltpu.sync_copy(ones_v, acc_sh.at[idx_a.at[1]], add=True)
    wait_idx(1, NCH - 1)
    pltpu.sync_copy(ones_v, acc_sh.at[idx_b.at[1]], add=True)

    plsc.subcore_barrier()
    pltpu.sync_copy(acc_sh.at[pl.ds(row0, SLAB)],
                    out_hbm.at[c, pl.ds(row0, SLAB)])


@functools.partial(
    pl.kernel,
    out_type=jax.ShapeDtypeStruct((NC, N_PAD, D), jnp.float32),
    mesh=_mesh,
    scratch_types=[
        pltpu.VMEM((2, C), jnp.int32),
        pltpu.VMEM((2, C), jnp.int32),
        pltpu.VMEM((C, D), jnp.float32),
        pltpu.VMEM((C, D), jnp.float32),
        pltpu.VMEM((32, D), jnp.float32),
        pltpu.VMEM_SHARED((N_PAD, D), jnp.float32),
        pltpu.SemaphoreType.DMA,
        pltpu.SemaphoreType.DMA,
        pltpu.SemaphoreType.DMA,
        pltpu.SemaphoreType.DMA,
    ],
)
def _sc_aggregate(g_hbm, idx_hbm, out_hbm,
                  idx_a, idx_b, rows_a, rows_b,
                  zb_v, acc_sh, semg_a, semg_b, semi_a, semi_b):
    c = lax.axis_index("c")
    s = lax.axis_index("s")
    wid = c * NS + s

    _zero_vmem(zb_v, 32, D)
    row0 = s * SLAB
    for k in range(SLAB // 32):
        pltpu.sync_copy(zb_v, acc_sh.at[pl.ds(row0 + k * 32, 32)])
    plsc.subcore_barrier()

    idx = (idx_a, idx_b)
    rows = (rows_a, rows_b)
    semg = (semg_a, semg_b)
    semi = (semi_a, semi_b)

    def issue_idx(slot, j):
        pltpu.async_copy(idx_hbm.at[wid, j], idx[slot], semi[slot])

    def wait_idx(slot, j):
        pltpu.make_async_copy(idx_hbm.at[wid, j], idx[slot],
                              semi[slot]).wait()

    def issue_gather(slot):
        pltpu.async_copy(g_hbm.at[idx[slot].at[0]], rows[slot], semg[slot])

    def wait_gather(slot):
        pltpu.make_async_copy(g_hbm.at[idx[slot].at[0]], rows[slot],
                              semg[slot]).wait()

    def scatter(slot):
        pltpu.sync_copy(rows[slot], acc_sh.at[idx[slot].at[1]], add=True)

    issue_idx(0, 0)
    issue_idx(1, 1)
    wait_idx(0, 0)
    issue_gather(0)

    def pair(j, _):
        for b in range(2):
            k = 2 * j + b
            nb = (b + 1) % 2
            wait_idx(nb, k + 1)
            issue_gather(nb)
            wait_gather(b)
            scatter(b)
            issue_idx(b, k + 2)
        return 0

    lax.fori_loop(0, NCH // 2 - 1, pair, 0)
    wait_idx(1, NCH - 1)
    issue_gather(1)
    wait_gather(0)
    scatter(0)
    wait_gather(1)
    scatter(1)

    plsc.subcore_barrier()
    pltpu.sync_copy(acc_sh.at[pl.ds(row0, SLAB)],
                    out_hbm.at[c, pl.ds(row0, SLAB)])


_R = 2000


def _gelu(z):
    return 0.5 * z * (1.0 + lax.erf(z * 0.7071067811865476))


def _tc_m_body(x_ref, w1_ref, h_ref):
    h_ref[...] = jnp.dot(x_ref[...], w1_ref[...],
                         preferred_element_type=jnp.float32)


def _tc_m(x, W1):
    return pl.pallas_call(
        _tc_m_body,
        grid=(N // _R,),
        in_specs=[
            pl.BlockSpec((_R, D), lambda i: (i, 0)),
            pl.BlockSpec((D, D), lambda i: (0, 0)),
        ],
        out_specs=pl.BlockSpec((_R, D), lambda i: (i, 0)),
        out_shape=jax.ShapeDtypeStruct((N, D), jnp.float32),
    )(x, W1)


def _tc_a_body(h_ref, degp_ref, g1_ref, dinvb_ref):
    d = degp_ref[0, :, 0:1] + degp_ref[1, :, 0:1] + 1.0
    dinv = lax.rsqrt(d)
    g1_ref[...] = h_ref[...] * dinv
    dinvb_ref[...] = jnp.broadcast_to(dinv, (_R, D))


def _tc_a(h1, degp):
    return pl.pallas_call(
        _tc_a_body,
        grid=(N // _R,),
        in_specs=[
            pl.BlockSpec((_R, D), lambda i: (i, 0)),
            pl.BlockSpec((NC, _R, D), lambda i: (0, i, 0)),
        ],
        out_specs=[
            pl.BlockSpec((_R, D), lambda i: (i, 0)),
            pl.BlockSpec((_R, D), lambda i: (i, 0)),
        ],
        out_shape=[
            jax.ShapeDtypeStruct((N, D), jnp.float32),
            jax.ShapeDtypeStruct((N, D), jnp.float32),
        ],
    )(h1, degp)


def _tc_b_body(p_ref, g1_ref, dinvb_ref, b1_ref, w2_ref, g2_ref):
    agg = p_ref[0] + p_ref[1] + g1_ref[...]
    z = agg * dinvb_ref[...] + b1_ref[...][None, :]
    a = _gelu(z)
    h2 = jnp.dot(a, w2_ref[...], preferred_element_type=jnp.float32)
    g2_ref[...] = h2 * dinvb_ref[...]


def _tc_b(p1, g1, dinvb, b1, W2):
    return pl.pallas_call(
        _tc_b_body,
        grid=(N // _R,),
        in_specs=[
            pl.BlockSpec((NC, _R, D), lambda i: (0, i, 0)),
            pl.BlockSpec((_R, D), lambda i: (i, 0)),
            pl.BlockSpec((_R, D), lambda i: (i, 0)),
            pl.BlockSpec((D,), lambda i: (0,)),
            pl.BlockSpec((D, D), lambda i: (0, 0)),
        ],
        out_specs=pl.BlockSpec((_R, D), lambda i: (i, 0)),
        out_shape=jax.ShapeDtypeStruct((N, D), jnp.float32),
    )(p1, g1, dinvb, b1, W2)


def _tc_c_body(p_ref, g2_ref, dinvb_ref, b2_ref, out_ref):
    agg = p_ref[0] + p_ref[1] + g2_ref[...]
    out_ref[...] = agg * dinvb_ref[...] + b2_ref[...][None, :]


def _tc_c(p2, g2, dinvb, b2):
    return pl.pallas_call(
        _tc_c_body,
        grid=(N // _R,),
        in_specs=[
            pl.BlockSpec((NC, _R, D), lambda i: (0, i, 0)),
            pl.BlockSpec((_R, D), lambda i: (i, 0)),
            pl.BlockSpec((_R, D), lambda i: (i, 0)),
            pl.BlockSpec((D,), lambda i: (0,)),
        ],
        out_specs=pl.BlockSpec((_R, D), lambda i: (i, 0)),
        out_shape=jax.ShapeDtypeStruct((N, D), jnp.float32),
    )(p2, g2, dinvb, b2)


def kernel(x, edge_index, W1, b1, W2, b2):
    pad = E_PAD - E
    pad_src = jnp.arange(pad, dtype=jnp.int32) % N
    pad_dst = N + jnp.arange(pad, dtype=jnp.int32) % (N_PAD - N)
    src_p = jnp.concatenate([edge_index[0], pad_src]).reshape(NW, NCH, C)
    dst_p = jnp.concatenate([edge_index[1], pad_dst]).reshape(NW, NCH, C)
    idx3 = jnp.stack([src_p, dst_p], axis=2)

    degp = _sc_degree(idx3)
    h1 = _tc_m(x, W1)
    g1, dinvb = _tc_a(h1, degp)
    p1 = _sc_aggregate(g1, idx3)
    g2 = _tc_b(p1, g1, dinvb, b1, W2)
    p2 = _sc_aggregate(g2, idx3)
    return _tc_c(p2, g2, dinvb, b2)


# speedup vs baseline: 1.0980x; 1.0031x over previous
import functools

import jax
import jax.numpy as jnp
from jax import lax
from jax.experimental import pallas as pl
from jax.experimental.pallas import tpu as pltpu
from jax.experimental.pallas import tpu_sc as plsc

N = 10000
E = 320000
D = 128

NC = 2
NS = 16
NW = NC * NS
C = 128
NCH = 80
E_PAD = NW * NCH * C
N_PAD = 10240
SLAB = N_PAD // NS

_mesh = plsc.VectorSubcoreMesh(core_axis_name="c", subcore_axis_name="s")


def _zero_vmem(ref, nrows, ncol):
    zeros = jnp.zeros((16,), jnp.float32)

    def body(i, _):
        for k in range(ncol // 16):
            ref[i, pl.ds(k * 16, 16)] = zeros
        return 0

    lax.fori_loop(0, nrows, body, 0)


@functools.partial(
    pl.kernel,
    out_type=jax.ShapeDtypeStruct((NC, N_PAD, D), jnp.float32),
    mesh=_mesh,
    scratch_types=[
        pltpu.VMEM((2, C), jnp.int32),
        pltpu.VMEM((2, C), jnp.int32),
        pltpu.VMEM((C, D), jnp.float32),
        pltpu.VMEM((32, D), jnp.float32),
        pltpu.VMEM_SHARED((N_PAD, D), jnp.float32),
        pltpu.SemaphoreType.DMA,
        pltpu.SemaphoreType.DMA,
    ],
)
def _sc_degree(idx_hbm, out_hbm, idx_a, idx_b, ones_v, zb_v, acc_sh,
               semi_a, semi_b):
    c = lax.axis_index("c")
    s = lax.axis_index("s")
    wid = c * NS + s
    idx = (idx_a, idx_b)
    semi = (semi_a, semi_b)

    one = jnp.full((16,), 1.0, jnp.float32)

    def issue_idx(slot, j):
        pltpu.async_copy(idx_hbm.at[wid, j], idx[slot], semi[slot])

    def wait_idx(slot, j):
        pltpu.make_async_copy(idx_hbm.at[wid, j], idx[slot],
                              semi[slot]).wait()

    issue_idx(0, 0)
    issue_idx(1, 1)

    def fill_ones(i, _):
        for k in range(D // 16):
            ones_v[i, pl.ds(k * 16, 16)] = one
        return 0

    lax.fori_loop(0, C, fill_ones, 0)
    _zero_vmem(zb_v, 32, D)
    row0 = s * SLAB
    for k in range(SLAB // 32):
        pltpu.sync_copy(zb_v, acc_sh.at[pl.ds(row0 + k * 32, 32)])
    plsc.subcore_barrier()

    def pair(j, _):
        for b in range(2):
            k = 2 * j + b
            wait_idx(b, k)
            pltpu.sync_copy(ones_v, acc_sh.at[idx[b].at[1]], add=True)
            issue_idx(b, k + 2)
        return 0

    lax.fori_loop(0, NCH // 2 - 1, pair, 0)
    wait_idx(0, NCH - 2)
    pltpu.sync_copy(ones_v, acc_sh.at[idx_a.at[1]], add=True)
    wait_idx(1, NCH - 1)
    pltpu.sync_copy(ones_v, acc_sh.at[idx_b.at[1]], add=True)

    plsc.subcore_barrier()
    pltpu.sync_copy(acc_sh.at[pl.ds(row0, SLAB)],
                    out_hbm.at[c, pl.ds(row0, SLAB)])


@functools.partial(
    pl.kernel,
    out_type=jax.ShapeDtypeStruct((NC, N_PAD, D), jnp.float32),
    mesh=_mesh,
    scratch_types=[
        pltpu.VMEM((2, C), jnp.int32),
        pltpu.VMEM((2, C), jnp.int32),
        pltpu.VMEM((C, D), jnp.float32),
        pltpu.VMEM((C, D), jnp.float32),
        pltpu.VMEM((32, D), jnp.float32),
        pltpu.VMEM_SHARED((N_PAD, D), jnp.float32),
        pltpu.SemaphoreType.DMA,
        pltpu.SemaphoreType.DMA,
        pltpu.SemaphoreType.DMA,
        pltpu.SemaphoreType.DMA,
    ],
)
def _sc_aggregate(g_hbm, idx_hbm, out_hbm,
                  idx_a, idx_b, rows_a, rows_b,
                  zb_v, acc_sh, semg_a, semg_b, semi_a, semi_b):
    c = lax.axis_index("c")
    s = lax.axis_index("s")
    wid = c * NS + s

    idx = (idx_a, idx_b)
    rows = (rows_a, rows_b)
    semg = (semg_a, semg_b)
    semi = (semi_a, semi_b)

    def issue_idx(slot, j):
        pltpu.async_copy(idx_hbm.at[wid, j], idx[slot], semi[slot])

    def wait_idx(slot, j):
        pltpu.make_async_copy(idx_hbm.at[wid, j], idx[slot],
                              semi[slot]).wait()

    def issue_gather(slot):
        pltpu.async_copy(g_hbm.at[idx[slot].at[0]], rows[slot], semg[slot])

    def wait_gather(slot):
        pltpu.make_async_copy(g_hbm.at[idx[slot].at[0]], rows[slot],
                              semg[slot]).wait()

    def scatter(slot):
        pltpu.sync_copy(rows[slot], acc_sh.at[idx[slot].at[1]], add=True)

    issue_idx(0, 0)
    issue_idx(1, 1)
    _zero_vmem(zb_v, 32, D)
    row0 = s * SLAB
    for k in range(SLAB // 32):
        pltpu.sync_copy(zb_v, acc_sh.at[pl.ds(row0 + k * 32, 32)])
    wait_idx(0, 0)
    issue_gather(0)
    plsc.subcore_barrier()

    def pair(j, _):
        for b in range(2):
            k = 2 * j + b
            nb = (b + 1) % 2
            wait_idx(nb, k + 1)
            issue_gather(nb)
            wait_gather(b)
            scatter(b)
            issue_idx(b, k + 2)
        return 0

    lax.fori_loop(0, NCH // 2 - 1, pair, 0)
    wait_idx(1, NCH - 1)
    issue_gather(1)
    wait_gather(0)
    scatter(0)
    wait_gather(1)
    scatter(1)

    plsc.subcore_barrier()
    pltpu.sync_copy(acc_sh.at[pl.ds(row0, SLAB)],
                    out_hbm.at[c, pl.ds(row0, SLAB)])


_R = 2000


def _gelu(z):
    return 0.5 * z * (1.0 + lax.erf(z * 0.7071067811865476))


def _tc_m_body(x_ref, w1_ref, h_ref):
    h_ref[...] = jnp.dot(x_ref[...], w1_ref[...],
                         preferred_element_type=jnp.float32)


def _tc_m(x, W1):
    return pl.pallas_call(
        _tc_m_body,
        grid=(N // _R,),
        in_specs=[
            pl.BlockSpec((_R, D), lambda i: (i, 0)),
            pl.BlockSpec((D, D), lambda i: (0, 0)),
        ],
        out_specs=pl.BlockSpec((_R, D), lambda i: (i, 0)),
        out_shape=jax.ShapeDtypeStruct((N, D), jnp.float32),
    )(x, W1)


def _tc_a_body(h_ref, degp_ref, g1_ref, dinvb_ref):
    d = degp_ref[0, :, 0:1] + degp_ref[1, :, 0:1] + 1.0
    dinv = lax.rsqrt(d)
    g1_ref[...] = h_ref[...] * dinv
    dinvb_ref[...] = jnp.broadcast_to(dinv, (_R, D))


def _tc_a(h1, degp):
    return pl.pallas_call(
        _tc_a_body,
        grid=(N // _R,),
        in_specs=[
            pl.BlockSpec((_R, D), lambda i: (i, 0)),
            pl.BlockSpec((NC, _R, D), lambda i: (0, i, 0)),
        ],
        out_specs=[
            pl.BlockSpec((_R, D), lambda i: (i, 0)),
            pl.BlockSpec((_R, D), lambda i: (i, 0)),
        ],
        out_shape=[
            jax.ShapeDtypeStruct((N, D), jnp.float32),
            jax.ShapeDtypeStruct((N, D), jnp.float32),
        ],
    )(h1, degp)


def _tc_b_body(p_ref, g1_ref, dinvb_ref, b1_ref, w2_ref, g2_ref):
    agg = p_ref[0] + p_ref[1] + g1_ref[...]
    z = agg * dinvb_ref[...] + b1_ref[...][None, :]
    a = _gelu(z)
    h2 = jnp.dot(a, w2_ref[...], preferred_element_type=jnp.float32)
    g2_ref[...] = h2 * dinvb_ref[...]


def _tc_b(p1, g1, dinvb, b1, W2):
    return pl.pallas_call(
        _tc_b_body,
        grid=(N // _R,),
        in_specs=[
            pl.BlockSpec((NC, _R, D), lambda i: (0, i, 0)),
            pl.BlockSpec((_R, D), lambda i: (i, 0)),
            pl.BlockSpec((_R, D), lambda i: (i, 0)),
            pl.BlockSpec((D,), lambda i: (0,)),
            pl.BlockSpec((D, D), lambda i: (0, 0)),
        ],
        out_specs=pl.BlockSpec((_R, D), lambda i: (i, 0)),
        out_shape=jax.ShapeDtypeStruct((N, D), jnp.float32),
    )(p1, g1, dinvb, b1, W2)


def _tc_c_body(p_ref, g2_ref, dinvb_ref, b2_ref, out_ref):
    agg = p_ref[0] + p_ref[1] + g2_ref[...]
    out_ref[...] = agg * dinvb_ref[...] + b2_ref[...][None, :]


def _tc_c(p2, g2, dinvb, b2):
    return pl.pallas_call(
        _tc_c_body,
        grid=(N // _R,),
        in_specs=[
            pl.BlockSpec((NC, _R, D), lambda i: (0, i, 0)),
            pl.BlockSpec((_R, D), lambda i: (i, 0)),
            pl.BlockSpec((_R, D), lambda i: (i, 0)),
            pl.BlockSpec((D,), lambda i: (0,)),
        ],
        out_specs=pl.BlockSpec((_R, D), lambda i: (i, 0)),
        out_shape=jax.ShapeDtypeStruct((N, D), jnp.float32),
    )(p2, g2, dinvb, b2)


def kernel(x, edge_index, W1, b1, W2, b2):
    pad = E_PAD - E
    pad_src = jnp.arange(pad, dtype=jnp.int32) % N
    pad_dst = N + jnp.arange(pad, dtype=jnp.int32) % (N_PAD - N)
    src_p = jnp.concatenate([edge_index[0], pad_src]).reshape(NW, NCH, C)
    dst_p = jnp.concatenate([edge_index[1], pad_dst]).reshape(NW, NCH, C)
    idx3 = jnp.stack([src_p, dst_p], axis=2)

    degp = _sc_degree(idx3)
    h1 = _tc_m(x, W1)
    g1, dinvb = _tc_a(h1, degp)
    p1 = _sc_aggregate(g1, idx3)
    g2 = _tc_b(p1, g1, dinvb, b1, W2)
    p2 = _sc_aggregate(g2, idx3)
    return _tc_c(p2, g2, dinvb, b2)
